# baseline (speedup 1.0000x reference)
import functools

import jax
import jax.numpy as jnp
from jax import lax
from jax.experimental import pallas as pl
from jax.experimental.pallas import tpu as pltpu
from jax.experimental.pallas import tpu_sc as plsc

VOCAB = 21
D = 64
NW = 32
K = 128


def _table_body(w_ref, b_ref, e_ref):
    eye = (lax.broadcasted_iota(jnp.int32, (VOCAB, VOCAB), 0)
           == lax.broadcasted_iota(jnp.int32, (VOCAB, VOCAB), 1)).astype(jnp.float32)
    e_ref[...] = lax.dot_general(
        eye, w_ref[...], (((1,), (1,)), ((), ())),
        preferred_element_type=jnp.float32) + b_ref[...]


def _make_table(W, b):
    return pl.pallas_call(
        _table_body,
        out_shape=jax.ShapeDtypeStruct((VOCAB, D), jnp.float32),
    )(W, b.reshape(1, D))


def _make_lookup(n_tokens):
    per_w = n_tokens // NW
    n_chunks = per_w // K
    mesh = plsc.VectorSubcoreMesh(core_axis_name="c", subcore_axis_name="s")

    @functools.partial(
        pl.kernel, mesh=mesh,
        compiler_params=pltpu.CompilerParams(use_tc_tiling_on_sc=False),
        out_type=jax.ShapeDtypeStruct((n_tokens, D), jnp.float32),
        scratch_types=[
            pltpu.VMEM((K,), jnp.int32),
            pltpu.VMEM((K, D), jnp.float32),
            pltpu.SemaphoreType.DMA,
        ],
    )
    def lookup(idx_hbm, table_hbm, out_hbm, idx_v, rows_v, sem):
        wid = lax.axis_index("s") * 2 + lax.axis_index("c")
        base = wid * per_w

        def body(g, carry):
            off = base + g * K
            pltpu.sync_copy(idx_hbm.at[pl.ds(off, K)], idx_v)
            pltpu.async_copy(table_hbm.at[idx_v], rows_v, sem).wait()
            pltpu.sync_copy(rows_v, out_hbm.at[pl.ds(off, K)])
            return carry

        lax.fori_loop(0, n_chunks, body, 0)

    return lookup


def kernel(idx, W, b):
    B, L = idx.shape
    n_tokens = B * L
    table = _make_table(W, b)
    flat_idx = idx.reshape(n_tokens).astype(jnp.int32)
    out = _make_lookup(n_tokens)(flat_idx, table)
    return out.reshape(B, L, D)


# speedup vs baseline: 1.0087x; 1.0087x over previous
import functools

import jax
import jax.numpy as jnp
from jax import lax
from jax.experimental import pallas as pl
from jax.experimental.pallas import tpu as pltpu
from jax.experimental.pallas import tpu_sc as plsc

VOCAB = 21
D = 64
NW = 32
K = 128
M = 4
CH = M * K
NBUF = 2


def _table_body(w_ref, b_ref, e_ref):
    eye = (lax.broadcasted_iota(jnp.int32, (VOCAB, VOCAB), 0)
           == lax.broadcasted_iota(jnp.int32, (VOCAB, VOCAB), 1)).astype(jnp.float32)
    e_ref[...] = lax.dot_general(
        eye, w_ref[...], (((1,), (1,)), ((), ())),
        preferred_element_type=jnp.float32) + b_ref[...]


def _make_table(W, b):
    return pl.pallas_call(
        _table_body,
        out_shape=jax.ShapeDtypeStruct((VOCAB, D), jnp.float32),
    )(W, b.reshape(1, D))


def _make_lookup(n_tokens):
    per_w = n_tokens // NW
    n_chunks = per_w // CH
    assert n_tokens == per_w * NW and per_w == n_chunks * CH
    assert n_chunks % NBUF == 0 and n_chunks >= 2 * NBUF
    mesh = plsc.VectorSubcoreMesh(core_axis_name="c", subcore_axis_name="s")

    @functools.partial(
        pl.kernel, mesh=mesh,
        compiler_params=pltpu.CompilerParams(use_tc_tiling_on_sc=False),
        out_type=jax.ShapeDtypeStruct((n_tokens, D), jnp.float32),
        scratch_types=[
            pltpu.VMEM((NBUF, M, K), jnp.int32),
            pltpu.VMEM((NBUF, CH, D), jnp.float32),
            pltpu.SemaphoreType.DMA,
            pltpu.SemaphoreType.DMA,
            pltpu.SemaphoreType.DMA,
            pltpu.SemaphoreType.DMA,
            pltpu.SemaphoreType.DMA,
            pltpu.SemaphoreType.DMA,
        ],
    )
    def lookup(idx_hbm, table_hbm, out_hbm, idx_v, rows_v,
               si0, si1, sg0, sg1, sw0, sw1):
        wid = lax.axis_index("s") * 2 + lax.axis_index("c")
        base = wid * per_w
        si = (si0, si1)
        sg = (sg0, sg1)
        sw = (sw0, sw1)

        def start_idx(g, b):
            off = base + g * CH
            for m in range(M):
                pltpu.async_copy(idx_hbm.at[pl.ds(off + m * K, K)],
                                 idx_v.at[b, m], si[b])

        def wait_idx(b):
            for m in range(M):
                pltpu.make_async_copy(idx_hbm.at[pl.ds(0, K)],
                                      idx_v.at[b, m], si[b]).wait()

        def do_gathers(b):
            descs = [
                pltpu.async_copy(table_hbm.at[idx_v.at[b, m]],
                                 rows_v.at[b, pl.ds(m * K, K)], sg[b])
                for m in range(M)
            ]
            for d in descs:
                d.wait()

        def start_write(g, b):
            off = base + g * CH
            pltpu.async_copy(rows_v.at[b], out_hbm.at[pl.ds(off, CH)], sw[b])

        def wait_write(b):
            pltpu.make_async_copy(rows_v.at[b], out_hbm.at[pl.ds(0, CH)],
                                  sw[b]).wait()

        for b in range(NBUF):
            start_idx(b, b)
        for b in range(NBUF):
            wait_idx(b)
            do_gathers(b)
            start_idx(b + NBUF, b)
            start_write(b, b)

        def body(o, carry):
            for b in range(NBUF):
                g = o * NBUF + b
                wait_write(b)
                wait_idx(b)
                do_gathers(b)
                start_idx(g + NBUF, b)
                start_write(g, b)
            return carry

        lax.fori_loop(1, n_chunks // NBUF - 1, body, 0)

        for b in range(NBUF):
            g = n_chunks - NBUF + b
            wait_write(b)
            wait_idx(b)
            do_gathers(b)
            start_write(g, b)
        for b in range(NBUF):
            wait_write(b)

    return lookup


def kernel(idx, W, b):
    B, L = idx.shape
    n_tokens = B * L
    table = _make_table(W, b)
    flat_idx = idx.reshape(n_tokens).astype(jnp.int32)
    out = _make_lookup(n_tokens)(flat_idx, table)
    return out.reshape(B, L, D)


# speedup vs baseline: 4.1439x; 4.1080x over previous
import functools

import jax
import jax.numpy as jnp
from jax import lax
from jax.experimental import pallas as pl
from jax.experimental.pallas import tpu as pltpu
from jax.experimental.pallas import tpu_sc as plsc

VOCAB = 21
D = 64
VPAD = 32
NW = 32
CH = 256
TU = 16
NBUF = 2
L16 = 16


def _table_body(w_ref, b_ref, e_ref):
    r = lax.broadcasted_iota(jnp.int32, (VPAD // 2, VOCAB), 0)
    k = lax.broadcasted_iota(jnp.int32, (VPAD // 2, VOCAB), 1)
    even = (2 * r == k).astype(jnp.float32)
    odd = (2 * r + 1 == k).astype(jnp.float32)
    dn = (((1,), (1,)), ((), ()))
    left = lax.dot_general(even, w_ref[...], dn,
                           preferred_element_type=jnp.float32) + b_ref[...]
    right = lax.dot_general(odd, w_ref[...], dn,
                            preferred_element_type=jnp.float32) + b_ref[...]
    e_ref[...] = jnp.concatenate([left, right], axis=1)


def _make_table(W, b):
    return pl.pallas_call(
        _table_body,
        out_shape=jax.ShapeDtypeStruct((VPAD * D // 128, 128), jnp.float32),
    )(W, b.reshape(1, D))


def _make_lookup(n_tokens):
    per_w = n_tokens // NW
    n_chunks = per_w // CH
    assert n_tokens == per_w * NW and per_w == n_chunks * CH
    assert n_chunks % NBUF == 0 and n_chunks >= 2 * NBUF and NBUF == 2
    mesh = plsc.VectorSubcoreMesh(core_axis_name="c", subcore_axis_name="s")

    @functools.partial(
        pl.kernel, mesh=mesh,
        compiler_params=pltpu.CompilerParams(needs_layout_passes=False),
        out_type=jax.ShapeDtypeStruct((n_tokens, 128), jnp.float32),
        scratch_types=[
            pltpu.VMEM((VPAD * D,), jnp.float32),
            pltpu.VMEM((CH,), jnp.int32),
            pltpu.VMEM((CH,), jnp.int32),
            pltpu.VMEM((CH, 128), jnp.float32),
            pltpu.VMEM((CH, 128), jnp.float32),
            pltpu.SemaphoreType.DMA,
            pltpu.SemaphoreType.DMA,
            pltpu.SemaphoreType.DMA,
            pltpu.SemaphoreType.DMA,
        ],
    )
    def lookup(idx_hbm, table_hbm, out_hbm, table_v, idxv0, idxv1,
               rows0, rows1, si0, si1, sw0, sw1):
        wid = lax.axis_index("s") * 2 + lax.axis_index("c")
        base = wid * per_w
        idxv = (idxv0, idxv1)
        rows = (rows0, rows1)
        si = (si0, si1)
        sw = (sw0, sw1)

        tdescs = [
            pltpu.async_copy(table_hbm.at[r0], table_v.at[pl.ds(r0 * 128, 128)],
                             si0)
            for r0 in range(VPAD * D // 128)
        ]
        for td in tdescs:
            td.wait()

        coff = [lax.iota(jnp.int32, L16) + j * L16 for j in range(4)]
        lane = [jnp.full((L16,), u, jnp.int32) for u in range(TU)]

        def start_idx(g, b):
            off = base + g * CH
            pltpu.async_copy(idx_hbm.at[pl.ds(off, CH)], idxv[b], si[b])

        def wait_idx(b):
            pltpu.make_async_copy(idx_hbm.at[pl.ds(0, CH)], idxv[b],
                                  si[b]).wait()

        def compute(b):
            rows_ref = rows[b]
            idx_ref = idxv[b]

            def step(it, carry):
                t0 = it * TU
                iv = idx_ref[pl.ds(t0, TU)]
                for u in range(TU):
                    t = t0 + u
                    sv = iv.at[lane[u]].get(mode="promise_in_bounds")
                    wbase = lax.shift_left(sv, 6)
                    for j in range(4):
                        words = wbase + coff[j]
                        vals = plsc.load_gather(table_v, [words])
                        rows_ref[t, pl.ds(j * L16, L16)] = vals
                return carry

            lax.fori_loop(0, CH // TU, step, 0)

        def start_write(g, b):
            off = base + g * CH
            pltpu.async_copy(rows[b], out_hbm.at[pl.ds(off, CH)], sw[b])

        def wait_write(b):
            pltpu.make_async_copy(rows[b], out_hbm.at[pl.ds(0, CH)],
                                  sw[b]).wait()

        for b in range(NBUF):
            start_idx(b, b)
        for b in range(NBUF):
            wait_idx(b)
            compute(b)
            start_idx(b + NBUF, b)
            start_write(b, b)

        def body(o, carry):
            for b in range(NBUF):
                g = o * NBUF + b
                wait_write(b)
                wait_idx(b)
                compute(b)
                start_idx(g + NBUF, b)
                start_write(g, b)
            return carry

        lax.fori_loop(1, n_chunks // NBUF - 1, body, 0)

        for b in range(NBUF):
            g = n_chunks - NBUF + b
            wait_write(b)
            wait_idx(b)
            compute(b)
            start_write(g, b)
        for b in range(NBUF):
            wait_write(b)

    return lookup


def kernel(idx, W, b):
    B, L = idx.shape
    n_tokens = B * L
    table = _make_table(W, b)
    flat_idx = idx.reshape(n_tokens).astype(jnp.int32)
    out = _make_lookup(n_tokens)(flat_idx, table)
    return out[:, :D].reshape(B, L, D)
